# baseline (speedup 1.0000x reference)
import functools

import jax
import jax.numpy as jnp
import numpy as np
from jax.experimental import pallas as pl
from jax.experimental.pallas import tpu as pltpu

_H = _W = 11
_HW = _H * _W
_C = 128


def _spiral_perm() -> np.ndarray:
    cen = _H // 2
    pos = [(cen, cen)]
    for r in range(1, cen + 1):
        pos += [(cen - r, w) for w in range(cen - r + 1, cen + r + 1)]
        pos += [(h, cen + r) for h in range(cen - r + 1, cen + r + 1)]
        pos += [(cen + r, w) for w in range(cen - r, cen + r)]
        pos += [(h, cen - r) for h in range(cen - r, cen + r)]
    return np.array([h * _W + w for h, w in pos], dtype=np.int64)


_P = np.zeros((_HW, _HW), dtype=np.float32)
_P[np.arange(_HW), _spiral_perm()] = 1.0


def _body(p_ref, x_ref, o_ref, *, bb):
    xb = x_ref[...]
    xm = xb.reshape(bb * _C, _HW)
    ym = jax.lax.dot_general(
        xm, p_ref[...],
        (((1,), (1,)), ((), ())),
        preferred_element_type=jnp.float32,
    )
    yb = ym.reshape(bb, _C, _HW)
    o_ref[...] = jnp.transpose(yb, (0, 2, 1))


@jax.jit
def kernel(x):
    b = x.shape[0]
    bb = 64
    xr = x.reshape(b, _C, _HW)
    return pl.pallas_call(
        functools.partial(_body, bb=bb),
        grid=(b // bb,),
        in_specs=[
            pl.BlockSpec((_HW, _HW), lambda i: (0, 0)),
            pl.BlockSpec((bb, _C, _HW), lambda i: (i, 0, 0)),
        ],
        out_specs=pl.BlockSpec((bb, _HW, _C), lambda i: (i, 0, 0)),
        out_shape=jax.ShapeDtypeStruct((b, _HW, _C), x.dtype),
        compiler_params=pltpu.CompilerParams(
            dimension_semantics=("parallel",),
        ),
    )(jnp.asarray(_P), xr)


# speedup vs baseline: 1.0058x; 1.0058x over previous
import functools

import jax
import jax.numpy as jnp
import numpy as np
from jax.experimental import pallas as pl
from jax.experimental.pallas import tpu as pltpu

_H = _W = 11
_HW = _H * _W
_C = 128


def _spiral_perm() -> np.ndarray:
    cen = _H // 2
    pos = [(cen, cen)]
    for r in range(1, cen + 1):
        pos += [(cen - r, w) for w in range(cen - r + 1, cen + r + 1)]
        pos += [(h, cen + r) for h in range(cen - r + 1, cen + r + 1)]
        pos += [(cen + r, w) for w in range(cen - r, cen + r)]
        pos += [(h, cen - r) for h in range(cen - r, cen + r)]
    return np.array([h * _W + w for h, w in pos], dtype=np.int64)


_P = np.zeros((_HW, _HW), dtype=np.float32)
_P[np.arange(_HW), _spiral_perm()] = 1.0


def _body(p_ref, x_ref, o_ref, *, bb):
    xb = x_ref[...]
    xm = xb.reshape(bb * _C, _HW)
    ym = jax.lax.dot_general(
        xm, p_ref[...],
        (((1,), (1,)), ((), ())),
        preferred_element_type=jnp.float32,
    )
    yb = ym.reshape(bb, _C, _HW)
    o_ref[...] = jnp.transpose(yb, (0, 2, 1))


@jax.jit
def kernel(x):
    b = x.shape[0]
    bb = 128
    xr = x.reshape(b, _C, _HW)
    return pl.pallas_call(
        functools.partial(_body, bb=bb),
        grid=(b // bb,),
        in_specs=[
            pl.BlockSpec((_HW, _HW), lambda i: (0, 0)),
            pl.BlockSpec((bb, _C, _HW), lambda i: (i, 0, 0)),
        ],
        out_specs=pl.BlockSpec((bb, _HW, _C), lambda i: (i, 0, 0)),
        out_shape=jax.ShapeDtypeStruct((b, _HW, _C), x.dtype),
        compiler_params=pltpu.CompilerParams(
            dimension_semantics=(pltpu.PARALLEL,),
        ),
    )(jnp.asarray(_P), xr)
